# Initial kernel scaffold; baseline (speedup 1.0000x reference)
import jax
import jax.numpy as jnp
from jax.experimental import pallas as pl


def kernel(x, edge_index, W1, b1, g1, be1, W2, b2, g2, be2, W3, b3, g3, be3):
    raise NotImplementedError("write your pallas kernel here")


# speedup vs baseline: 10.5105x; 10.5105x over previous
import functools
import math

import jax
import jax.numpy as jnp
from jax import lax
from jax.experimental import pallas as pl
from jax.experimental.pallas import tpu as pltpu
from jax.experimental.pallas import tpu_sc as plsc

_NC = 2
_NS = 16
_BN_C = 1.0 / math.sqrt(1.0 + 1e-5)


def _row_init(sid, src, dst, n, rows_pt, rows_tail):
  base_r = sid * rows_pt
  pltpu.sync_copy(src.at[pl.ds(base_r, rows_pt)], dst.at[pl.ds(base_r, rows_pt)])
  if rows_tail:
    @pl.when(sid == _NS - 1)
    def _tail():
      pltpu.sync_copy(src.at[pl.ds(rows_pt * _NS, rows_tail)],
                      dst.at[pl.ds(rows_pt * _NS, rows_tail)])


def _splits(n, d, e):
  nw = _NC * _NS
  epw = e // nw
  chunk = 80
  nchunks = epw // chunk
  assert epw * nw == e and nchunks * chunk == epw and d % 128 == 0
  rows_pt = (n // _NS) & ~7
  rows_tail = n - rows_pt * _NS
  assert rows_tail >= 0 and rows_tail % 8 == 0
  return epw, chunk, nchunks, rows_pt, rows_tail


@functools.lru_cache(maxsize=None)
def _make_agg(n, d, e):
  epw, chunk, nchunks, rows_pt, rows_tail = _splits(n, d, e)
  mesh = plsc.VectorSubcoreMesh(core_axis_name="c", subcore_axis_name="s")

  @functools.partial(
      pl.kernel,
      out_type=jax.ShapeDtypeStruct((_NC, n, d), jnp.float32),
      mesh=mesh,
      scratch_types=[
          pltpu.VMEM((chunk,), jnp.int32),
          pltpu.VMEM((chunk,), jnp.int32),
          pltpu.VMEM((chunk, d), jnp.float32),
          pltpu.VMEM_SHARED((n, d), jnp.float32),
          pltpu.SemaphoreType.DMA,
      ],
  )
  def agg(u_hbm, row_hbm, col_hbm, out_hbm, row_v, col_v, gbuf, acc, sem):
    cid = lax.axis_index("c")
    sid = lax.axis_index("s")
    wid = sid * _NC + cid
    _row_init(sid, u_hbm, acc, n, rows_pt, rows_tail)
    plsc.subcore_barrier()
    base_e = wid * epw

    def body(i, carry):
      b = base_e + i * chunk
      pltpu.sync_copy(row_hbm.at[pl.ds(b, chunk)], row_v)
      pltpu.sync_copy(col_hbm.at[pl.ds(b, chunk)], col_v)
      pltpu.async_copy(u_hbm.at[row_v], gbuf, sem).wait()
      pltpu.sync_copy(gbuf, acc.at[col_v], add=True)
      return carry

    lax.fori_loop(0, nchunks, body, 0)
    plsc.subcore_barrier()
    _row_init(sid, acc, out_hbm.at[cid], n, rows_pt, rows_tail)

  return agg


@functools.lru_cache(maxsize=None)
def _make_deg(n, d, e):
  epw, chunk, nchunks, rows_pt, rows_tail = _splits(n, d, e)
  mesh = plsc.VectorSubcoreMesh(core_axis_name="c", subcore_axis_name="s")

  @functools.partial(
      pl.kernel,
      out_type=jax.ShapeDtypeStruct((_NC, n, d), jnp.float32),
      mesh=mesh,
      scratch_types=[
          pltpu.VMEM((chunk,), jnp.int32),
          pltpu.VMEM((chunk, d), jnp.float32),
          pltpu.VMEM_SHARED((n, d), jnp.float32),
      ],
  )
  def deg(half_hbm, ones_hbm, col_hbm, out_hbm, col_v, ones_v, acc):
    cid = lax.axis_index("c")
    sid = lax.axis_index("s")
    wid = sid * _NC + cid
    _row_init(sid, half_hbm, acc, n, rows_pt, rows_tail)
    pltpu.sync_copy(ones_hbm.at[pl.ds(0, chunk)], ones_v)
    plsc.subcore_barrier()
    base_e = wid * epw

    def body(i, carry):
      pltpu.sync_copy(col_hbm.at[pl.ds(base_e + i * chunk, chunk)], col_v)
      pltpu.sync_copy(ones_v, acc.at[col_v], add=True)
      return carry

    lax.fori_loop(0, nchunks, body, 0)
    plsc.subcore_barrier()
    _row_init(sid, acc, out_hbm.at[cid], n, rows_pt, rows_tail)

  return deg


def _first_body(x, d0, d1, w, o_u, o_dis):
  dis = lax.rsqrt(d0[...] + d1[...])
  o_dis[...] = dis
  o_u[...] = jnp.dot(x[...], w[...], preferred_element_type=jnp.float32) * dis


def _mid_body(a0, a1, up, dis, b, g, be, w, o):
  d = dis[...]
  z = (a0[...] + a1[...] - up[...]) * d + b[...]
  z = z * (g[...] * _BN_C) + be[...]
  z = jnp.maximum(z, 0.0)
  o[...] = jnp.dot(z, w[...], preferred_element_type=jnp.float32) * d


def _last_body(a0, a1, up, dis, b, g, be, o):
  z = (a0[...] + a1[...] - up[...]) * dis[...] + b[...]
  o[...] = z * (g[...] * _BN_C) + be[...]


def _tc_call(body, n, d, r, arrs, vecs, weights, num_out=1):
  grid = (n // r,)
  nd_spec = pl.BlockSpec((r, d), lambda i: (i, 0))
  vec_spec = pl.BlockSpec((1, d), lambda i: (0, 0))
  w_spec = pl.BlockSpec((d, d), lambda i: (0, 0))
  in_specs = ([nd_spec] * len(arrs) + [vec_spec] * len(vecs)
              + [w_spec] * len(weights))
  shape = jax.ShapeDtypeStruct((n, d), jnp.float32)
  return pl.pallas_call(
      body,
      grid=grid,
      in_specs=in_specs,
      out_specs=[nd_spec] * num_out if num_out > 1 else nd_spec,
      out_shape=[shape] * num_out if num_out > 1 else shape,
  )(*arrs, *vecs, *weights)


def kernel(x, edge_index, W1, b1, g1, be1, W2, b2, g2, be2, W3, b3, g3, be3):
  n, d = x.shape
  e = edge_index.shape[1]
  row = edge_index[0]
  col = edge_index[1]

  halves = jnp.full((n, d), 0.5, jnp.float32)
  ones = jnp.ones((80, d), jnp.float32)
  deg_pair = _make_deg(n, d, e)(halves, ones, col)

  agg = _make_agg(n, d, e)
  r = 1000

  u1, disb = _tc_call(_first_body, n, d, r, (x, deg_pair[0], deg_pair[1]),
                      (), (W1,), num_out=2)
  a1 = agg(u1, row, col)
  u2 = _tc_call(_mid_body, n, d, r, (a1[0], a1[1], u1, disb),
                (b1.reshape(1, d), g1.reshape(1, d), be1.reshape(1, d)), (W2,))
  a2 = agg(u2, row, col)
  u3 = _tc_call(_mid_body, n, d, r, (a2[0], a2[1], u2, disb),
                (b2.reshape(1, d), g2.reshape(1, d), be2.reshape(1, d)), (W3,))
  a3 = agg(u3, row, col)
  out = _tc_call(_last_body, n, d, r, (a3[0], a3[1], u3, disb),
                 (b3.reshape(1, d), g3.reshape(1, d), be3.reshape(1, d)), ())
  return out


# speedup vs baseline: 18.8416x; 1.7926x over previous
import functools
import math

import jax
import jax.numpy as jnp
from jax import lax
from jax.experimental import pallas as pl
from jax.experimental.pallas import tpu as pltpu
from jax.experimental.pallas import tpu_sc as plsc

_NC = 2
_NS = 16
_NW = _NC * _NS
_CHUNK = 80
_BN_C = 1.0 / math.sqrt(1.0 + 1e-5)


def _row_init(sid, src, dst, rows_pt, rows_tail):
  base_r = sid * rows_pt
  pltpu.sync_copy(src.at[pl.ds(base_r, rows_pt)], dst.at[pl.ds(base_r, rows_pt)])
  if rows_tail:
    @pl.when(sid == _NS - 1)
    def _tail():
      pltpu.sync_copy(src.at[pl.ds(rows_pt * _NS, rows_tail)],
                      dst.at[pl.ds(rows_pt * _NS, rows_tail)])


def _splits(n, d, e):
  epw = e // _NW
  nchunks = epw // _CHUNK
  assert epw * _NW == e and nchunks * _CHUNK == epw and d % 128 == 0
  rows_pt = (n // _NS) & ~7
  rows_tail = n - rows_pt * _NS
  assert rows_tail >= 0 and rows_tail % 8 == 0
  return nchunks, rows_pt, rows_tail


@functools.lru_cache(maxsize=None)
def _make_agg(n, d, e):
  nchunks, rows_pt, rows_tail = _splits(n, d, e)
  npair = (nchunks - 1) // 2
  assert npair * 2 + 1 == nchunks
  mesh = plsc.VectorSubcoreMesh(core_axis_name="c", subcore_axis_name="s")

  @functools.partial(
      pl.kernel,
      out_type=jax.ShapeDtypeStruct((_NC, n, d), jnp.float32),
      mesh=mesh,
      scratch_types=[
          pltpu.VMEM((nchunks * _CHUNK,), jnp.int32),
          pltpu.VMEM((nchunks, _CHUNK), jnp.int32),
          pltpu.VMEM((2, _CHUNK, d), jnp.float32),
          pltpu.VMEM_SHARED((n, d), jnp.float32),
          pltpu.SemaphoreType.DMA,
          pltpu.SemaphoreType.DMA,
          pltpu.SemaphoreType.DMA,
          pltpu.SemaphoreType.DMA,
      ],
  )
  def agg(u_hbm, row2_hbm, col3_hbm, out_hbm, row_v, col_v, gbuf, acc,
          gsem0, gsem1, ssem0, ssem1):
    cid = lax.axis_index("c")
    sid = lax.axis_index("s")
    wid = sid * _NC + cid
    gb = (gbuf.at[0], gbuf.at[1])
    gsems = (gsem0, gsem1)
    ssems = (ssem0, ssem1)

    pltpu.sync_copy(row2_hbm.at[wid], row_v)
    pltpu.sync_copy(col3_hbm.at[wid], col_v)
    _row_init(sid, u_hbm, acc, rows_pt, rows_tail)
    plsc.subcore_barrier()

    def start_gather(ci, p):
      pltpu.async_copy(u_hbm.at[row_v.at[pl.ds(ci * _CHUNK, _CHUNK)]],
                       gb[p], gsems[p])

    def wait_gather(p):
      pltpu.make_async_copy(u_hbm.at[row_v.at[pl.ds(0, _CHUNK)]],
                            gb[p], gsems[p]).wait()

    def start_scatter(ci, p):
      pltpu.async_copy(gb[p], acc.at[col_v.at[ci]], ssems[p], add=True)

    def wait_scatter(p):
      pltpu.make_async_copy(gb[p], acc.at[col_v.at[0]], ssems[p]).wait()

    start_gather(0, 0)
    start_gather(1, 1)

    def body(k, carry):
      c0 = 2 * k
      wait_gather(0)
      start_scatter(c0, 0)
      wait_gather(1)
      start_scatter(c0 + 1, 1)
      wait_scatter(0)
      start_gather(c0 + 2, 0)
      wait_scatter(1)

      @pl.when(k < npair - 1)
      def _pref():
        start_gather(c0 + 3, 1)

      return carry

    lax.fori_loop(0, npair, body, 0)
    wait_gather(0)
    start_scatter(nchunks - 1, 0)
    wait_scatter(0)
    plsc.subcore_barrier()
    _row_init(sid, acc, out_hbm.at[cid], rows_pt, rows_tail)

  return agg


@functools.lru_cache(maxsize=None)
def _make_deg(n, d, e):
  nchunks, rows_pt, rows_tail = _splits(n, d, e)
  window = 8
  mesh = plsc.VectorSubcoreMesh(core_axis_name="c", subcore_axis_name="s")

  @functools.partial(
      pl.kernel,
      out_type=jax.ShapeDtypeStruct((_NC, n, d), jnp.float32),
      mesh=mesh,
      scratch_types=[
          pltpu.VMEM((nchunks, _CHUNK), jnp.int32),
          pltpu.VMEM((_CHUNK, d), jnp.float32),
          pltpu.VMEM_SHARED((n, d), jnp.float32),
          pltpu.SemaphoreType.DMA,
      ],
  )
  def deg(half_hbm, ones_hbm, col3_hbm, out_hbm, col_v, ones_v, acc, ssem):
    cid = lax.axis_index("c")
    sid = lax.axis_index("s")
    wid = sid * _NC + cid
    pltpu.sync_copy(col3_hbm.at[wid], col_v)
    pltpu.sync_copy(ones_hbm.at[pl.ds(0, _CHUNK)], ones_v)
    _row_init(sid, half_hbm, acc, rows_pt, rows_tail)
    plsc.subcore_barrier()

    def start_sc(ci):
      pltpu.async_copy(ones_v, acc.at[col_v.at[ci]], ssem, add=True)

    def wait_sc(i, carry):
      pltpu.make_async_copy(ones_v, acc.at[col_v.at[0]], ssem).wait()
      return carry

    for ci in range(window):
      start_sc(ci)

    def body(k, carry):
      wait_sc(k, carry)
      start_sc(k + window)
      return carry

    lax.fori_loop(0, nchunks - window, body, 0)
    lax.fori_loop(0, window, wait_sc, 0)
    plsc.subcore_barrier()
    _row_init(sid, acc, out_hbm.at[cid], rows_pt, rows_tail)

  return deg


def _first_body(x, d0, d1, w, o_u, o_dis):
  dis = lax.rsqrt(d0[...] + d1[...])
  o_dis[...] = dis
  o_u[...] = jnp.dot(x[...], w[...], preferred_element_type=jnp.float32) * dis


def _mid_body(a0, a1, up, dis, b, g, be, w, o):
  d = dis[...]
  z = (a0[...] + a1[...] - up[...]) * d + b[...]
  z = z * (g[...] * _BN_C) + be[...]
  z = jnp.maximum(z, 0.0)
  o[...] = jnp.dot(z, w[...], preferred_element_type=jnp.float32) * d


def _last_body(a0, a1, up, dis, b, g, be, o):
  z = (a0[...] + a1[...] - up[...]) * dis[...] + b[...]
  o[...] = z * (g[...] * _BN_C) + be[...]


def _tc_call(body, n, d, r, arrs, vecs, weights, num_out=1):
  grid = (n // r,)
  nd_spec = pl.BlockSpec((r, d), lambda i: (i, 0))
  vec_spec = pl.BlockSpec((1, d), lambda i: (0, 0))
  w_spec = pl.BlockSpec((d, d), lambda i: (0, 0))
  in_specs = ([nd_spec] * len(arrs) + [vec_spec] * len(vecs)
              + [w_spec] * len(weights))
  shape = jax.ShapeDtypeStruct((n, d), jnp.float32)
  return pl.pallas_call(
      body,
      grid=grid,
      in_specs=in_specs,
      out_specs=[nd_spec] * num_out if num_out > 1 else nd_spec,
      out_shape=[shape] * num_out if num_out > 1 else shape,
  )(*arrs, *vecs, *weights)


def kernel(x, edge_index, W1, b1, g1, be1, W2, b2, g2, be2, W3, b3, g3, be3):
  n, d = x.shape
  e = edge_index.shape[1]
  nchunks = e // (_NW * _CHUNK)
  row2 = edge_index[0].reshape(_NW, nchunks * _CHUNK)
  col3 = edge_index[1].reshape(_NW, nchunks, _CHUNK)

  halves = jnp.full((n, d), 0.5, jnp.float32)
  ones = jnp.ones((_CHUNK, d), jnp.float32)
  deg_pair = _make_deg(n, d, e)(halves, ones, col3)

  agg = _make_agg(n, d, e)
  r = 1000

  u1, disb = _tc_call(_first_body, n, d, r, (x, deg_pair[0], deg_pair[1]),
                      (), (W1,), num_out=2)
  a1 = agg(u1, row2, col3)
  u2 = _tc_call(_mid_body, n, d, r, (a1[0], a1[1], u1, disb),
                (b1.reshape(1, d), g1.reshape(1, d), be1.reshape(1, d)), (W2,))
  a2 = agg(u2, row2, col3)
  u3 = _tc_call(_mid_body, n, d, r, (a2[0], a2[1], u2, disb),
                (b2.reshape(1, d), g2.reshape(1, d), be2.reshape(1, d)), (W3,))
  a3 = agg(u3, row2, col3)
  out = _tc_call(_last_body, n, d, r, (a3[0], a3[1], u3, disb),
                 (b3.reshape(1, d), g3.reshape(1, d), be3.reshape(1, d)), ())
  return out


# speedup vs baseline: 26.2480x; 1.3931x over previous
import functools
import math

import jax
import jax.numpy as jnp
from jax import lax
from jax.experimental import pallas as pl
from jax.experimental.pallas import tpu as pltpu
from jax.experimental.pallas import tpu_sc as plsc

_NC = 2
_NS = 16
_NW = _NC * _NS
_CHUNK = 80
_BN_C = 1.0 / math.sqrt(1.0 + 1e-5)


def _row_init(sid, src, dst, rows_pt, rows_tail):
  base_r = sid * rows_pt
  pltpu.sync_copy(src.at[pl.ds(base_r, rows_pt)], dst.at[pl.ds(base_r, rows_pt)])
  if rows_tail:
    @pl.when(sid == _NS - 1)
    def _tail():
      pltpu.sync_copy(src.at[pl.ds(rows_pt * _NS, rows_tail)],
                      dst.at[pl.ds(rows_pt * _NS, rows_tail)])


def _splits(n, d, e):
  epw = e // _NW
  nchunks = epw // _CHUNK
  assert epw * _NW == e and nchunks * _CHUNK == epw and d % 128 == 0
  rows_pt = (n // _NS) & ~7
  rows_tail = n - rows_pt * _NS
  assert rows_tail >= 0 and rows_tail % 8 == 0
  return nchunks, rows_pt, rows_tail


@functools.lru_cache(maxsize=None)
def _make_agg(n, d, e):
  nchunks, rows_pt, rows_tail = _splits(n, d, e)
  nmain = (nchunks // 8) * 8
  tail = list(range(nmain, nchunks))
  assert len(tail) <= 6
  mesh = plsc.VectorSubcoreMesh(core_axis_name="c", subcore_axis_name="s")

  @functools.partial(
      pl.kernel,
      out_type=jax.ShapeDtypeStruct((_NC, n, d), jnp.float32),
      mesh=mesh,
      scratch_types=[
          pltpu.VMEM((8, _CHUNK), jnp.int32),
          pltpu.VMEM((8, _CHUNK), jnp.int32),
          pltpu.VMEM((4, _CHUNK, d), jnp.float32),
          pltpu.VMEM_SHARED((n, d), jnp.float32),
      ] + [pltpu.SemaphoreType.DMA] * 16,
  )
  def agg(u_hbm, row1_hbm, col1_hbm, out_hbm, rowring, colring, gbuf, acc,
          *sems):
    gsem = sems[0:4]
    ssem = sems[4:8]
    isem = sems[8:16]
    cid = lax.axis_index("c")
    sid = lax.axis_index("s")
    wid = sid * _NC + cid
    base_e = wid * (nchunks * _CHUNK)
    _row_init(sid, u_hbm, acc, rows_pt, rows_tail)

    def start_fetch(ci, i8):
      pltpu.async_copy(row1_hbm.at[pl.ds(base_e + ci * _CHUNK, _CHUNK)],
                       rowring.at[i8], isem[i8])
      pltpu.async_copy(col1_hbm.at[pl.ds(base_e + ci * _CHUNK, _CHUNK)],
                       colring.at[i8], isem[i8])

    def wait_fetch(i8):
      pltpu.make_async_copy(row1_hbm.at[pl.ds(0, _CHUNK)],
                            rowring.at[i8], isem[i8]).wait()
      pltpu.make_async_copy(row1_hbm.at[pl.ds(0, _CHUNK)],
                            colring.at[i8], isem[i8]).wait()

    def start_gather(i8, g):
      pltpu.async_copy(u_hbm.at[rowring.at[i8]], gbuf.at[g], gsem[g])

    def wait_gather(g):
      pltpu.make_async_copy(u_hbm.at[rowring.at[0]], gbuf.at[g],
                            gsem[g]).wait()

    def start_scatter(i8, g):
      pltpu.async_copy(gbuf.at[g], acc.at[colring.at[i8]], ssem[g], add=True)

    def wait_scatter(g):
      pltpu.make_async_copy(gbuf.at[g], acc.at[colring.at[0]], ssem[g]).wait()

    for c in range(4):
      start_fetch(c, c)
    plsc.subcore_barrier()

    def body(k, carry):
      c0 = 8 * k
      for b in range(8):
        ci = c0 + b

        @pl.when(ci >= 4)
        def _recycle():
          wait_scatter(b % 4)

        @pl.when(ci + 4 < nmain)
        def _pref():
          start_fetch(ci + 4, (b + 4) % 8)

        wait_fetch(b)
        start_gather(b, b % 4)

        if b >= 2:
          wait_gather((b - 2) % 4)
          start_scatter((b - 2) % 8, (b - 2) % 4)
        else:
          @pl.when(ci >= 2)
          def _scat():
            wait_gather((b - 2) % 4)
            start_scatter((b - 2) % 8, (b - 2) % 4)

      return carry

    lax.fori_loop(0, nmain // 8, body, 0)
    for ci in tail:
      wait_scatter(ci % 4)
      start_fetch(ci, ci % 8)
      wait_fetch(ci % 8)
      start_gather(ci % 8, ci % 4)
      wait_gather((ci - 2) % 4)
      start_scatter((ci - 2) % 8, (ci - 2) % 4)
    for ci in (nchunks - 2, nchunks - 1):
      wait_gather(ci % 4)
      start_scatter(ci % 8, ci % 4)
    for ci in range(nchunks - 4, nchunks):
      wait_scatter(ci % 4)
    plsc.subcore_barrier()
    _row_init(sid, acc, out_hbm.at[cid], rows_pt, rows_tail)

  return agg


_DEGW = 64


@functools.lru_cache(maxsize=None)
def _make_deg(n, d, e):
  nchunks, rows_pt, rows_tail = _splits(n, d, e)
  window = 8
  mesh = plsc.VectorSubcoreMesh(core_axis_name="c", subcore_axis_name="s")

  @functools.partial(
      pl.kernel,
      out_type=jax.ShapeDtypeStruct((_NC, n, _DEGW), jnp.float32),
      mesh=mesh,
      scratch_types=[
          pltpu.VMEM((nchunks, _CHUNK), jnp.int32),
          pltpu.VMEM((_CHUNK, _DEGW), jnp.float32),
          pltpu.VMEM_SHARED((n, _DEGW), jnp.float32),
          pltpu.SemaphoreType.DMA,
      ],
  )
  def deg(half_hbm, ones_hbm, col3_hbm, out_hbm, col_v, ones_v, acc, ssem):
    cid = lax.axis_index("c")
    sid = lax.axis_index("s")
    wid = sid * _NC + cid
    pltpu.sync_copy(col3_hbm.at[wid], col_v)
    pltpu.sync_copy(ones_hbm.at[pl.ds(0, _CHUNK)], ones_v)
    _row_init(sid, half_hbm, acc, rows_pt, rows_tail)
    plsc.subcore_barrier()

    def start_sc(ci):
      pltpu.async_copy(ones_v, acc.at[col_v.at[ci]], ssem, add=True)

    def wait_sc(i, carry):
      pltpu.make_async_copy(ones_v, acc.at[col_v.at[0]], ssem).wait()
      return carry

    for ci in range(window):
      start_sc(ci)

    def body(k, carry):
      wait_sc(k, carry)
      start_sc(k + window)
      return carry

    lax.fori_loop(0, nchunks - window, body, 0)
    lax.fori_loop(0, window, wait_sc, 0)
    plsc.subcore_barrier()
    _row_init(sid, acc, out_hbm.at[cid], rows_pt, rows_tail)

  return deg


def _first_body(x, d0, d1, w, o_u, o_dis):
  deg = d0[...][:, :1] + d1[...][:, :1]
  dis = jnp.broadcast_to(lax.rsqrt(deg), o_dis.shape)
  o_dis[...] = dis
  o_u[...] = jnp.dot(x[...], w[...], preferred_element_type=jnp.float32) * dis


def _mid_body(a0, a1, up, dis, b, g, be, w, o):
  d = dis[...]
  z = (a0[...] + a1[...] - up[...]) * d + b[...]
  z = z * (g[...] * _BN_C) + be[...]
  z = jnp.maximum(z, 0.0)
  o[...] = jnp.dot(z, w[...], preferred_element_type=jnp.float32) * d


def _last_body(a0, a1, up, dis, b, g, be, o):
  z = (a0[...] + a1[...] - up[...]) * dis[...] + b[...]
  o[...] = z * (g[...] * _BN_C) + be[...]


def _tc_call(body, n, d, r, arrs, vecs, weights, num_out=1, num_narrow=0):
  grid = (n // r,)
  nd_spec = pl.BlockSpec((r, d), lambda i: (i, 0))
  nr_spec = pl.BlockSpec((r, _DEGW), lambda i: (i, 0))
  vec_spec = pl.BlockSpec((1, d), lambda i: (0, 0))
  w_spec = pl.BlockSpec((d, d), lambda i: (0, 0))
  in_specs = ([nd_spec] * (len(arrs) - num_narrow) + [nr_spec] * num_narrow
              + [vec_spec] * len(vecs) + [w_spec] * len(weights))
  shape = jax.ShapeDtypeStruct((n, d), jnp.float32)
  return pl.pallas_call(
      body,
      grid=grid,
      in_specs=in_specs,
      out_specs=[nd_spec] * num_out if num_out > 1 else nd_spec,
      out_shape=[shape] * num_out if num_out > 1 else shape,
  )(*arrs, *vecs, *weights)


def kernel(x, edge_index, W1, b1, g1, be1, W2, b2, g2, be2, W3, b3, g3, be3):
  n, d = x.shape
  e = edge_index.shape[1]
  nchunks = e // (_NW * _CHUNK)
  row1 = edge_index[0]
  col1 = edge_index[1]
  col3 = edge_index[1].reshape(_NW, nchunks, _CHUNK)

  halves = jnp.full((n, _DEGW), 0.5, jnp.float32)
  ones = jnp.ones((_CHUNK, _DEGW), jnp.float32)
  deg_pair = _make_deg(n, d, e)(halves, ones, col3)

  agg = _make_agg(n, d, e)
  r = 1000

  u1, disb = _tc_call(_first_body, n, d, r, (x, deg_pair[0], deg_pair[1]),
                      (), (W1,), num_out=2, num_narrow=2)
  a1 = agg(u1, row1, col1)
  u2 = _tc_call(_mid_body, n, d, r, (a1[0], a1[1], u1, disb),
                (b1.reshape(1, d), g1.reshape(1, d), be1.reshape(1, d)), (W2,))
  a2 = agg(u2, row1, col1)
  u3 = _tc_call(_mid_body, n, d, r, (a2[0], a2[1], u2, disb),
                (b2.reshape(1, d), g2.reshape(1, d), be2.reshape(1, d)), (W3,))
  a3 = agg(u3, row1, col1)
  out = _tc_call(_last_body, n, d, r, (a3[0], a3[1], u3, disb),
                 (b3.reshape(1, d), g3.reshape(1, d), be3.reshape(1, d)), ())
  return out
